# baseline (speedup 1.0000x reference)
import functools

import jax
import jax.numpy as jnp
from jax import lax
from jax.experimental import pallas as pl
from jax.experimental.pallas import tpu as pltpu
from jax.experimental.pallas import tpu_sc as plsc

NC, NS, L = 2, 16, 16
NW = NC * NS
C = 80


@functools.lru_cache(maxsize=None)
def _build(B, V, D):
    BPW = B // NW
    NCH = BPW // C
    assert BPW * NW == B and NCH * C == BPW and NCH % 2 == 0

    mesh = plsc.VectorSubcoreMesh(core_axis_name="c", subcore_axis_name="s")

    @functools.partial(
        pl.kernel,
        out_type=jax.ShapeDtypeStruct((B, D), jnp.float32),
        mesh=mesh,
        compiler_params=pltpu.CompilerParams(needs_layout_passes=False),
        scratch_types=[
            pltpu.VMEM((BPW,), jnp.int32),
            pltpu.VMEM((C, D), jnp.float32),
            pltpu.VMEM((C, D), jnp.float32),
            pltpu.SemaphoreType.DMA,
            pltpu.SemaphoreType.DMA,
            pltpu.SemaphoreType.DMA,
            pltpu.SemaphoreType.DMA,
        ],
    )
    def emb(idx_hbm, table_hbm, out_hbm, idx_v, rows0, rows1, g0, g1, o0, o1):
        rows = (rows0, rows1)
        gsem = (g0, g1)
        osem = (o0, o1)
        wid = lax.axis_index("s") * NC + lax.axis_index("c")
        base = wid * BPW
        pltpu.sync_copy(idx_hbm.at[pl.ds(base, BPW)], idx_v)

        def start_g(c, b):
            pltpu.async_copy(table_hbm.at[idx_v.at[pl.ds(c * C, C)]],
                             rows[b], gsem[b])

        def wait_g(b):
            pltpu.make_async_copy(table_hbm.at[idx_v.at[pl.ds(0, C)]],
                                  rows[b], gsem[b]).wait()

        def start_o(c, b):
            pltpu.async_copy(rows[b], out_hbm.at[pl.ds(base + c * C, C)],
                             osem[b])

        def wait_o(b):
            pltpu.make_async_copy(rows[b], out_hbm.at[pl.ds(0, C)],
                                  osem[b]).wait()

        start_g(0, 0)

        def step(c, b):
            wait_g(b)
            start_o(c, b)

            @pl.when(c + 1 < NCH)
            def _():
                @pl.when(c >= 1)
                def _():
                    wait_o(1 - b)
                start_g(c + 1, 1 - b)

        def body(i, carry):
            step(2 * i, 0)
            step(2 * i + 1, 1)
            return carry

        lax.fori_loop(0, NCH // 2, body, 0)
        wait_o(0)
        wait_o(1)

    return emb


def kernel(word_seq, embedding_table):
    s, t = word_seq.shape
    b = s * t
    v, d = embedding_table.shape
    bpw = b // NW
    idx = word_seq.reshape(NW, bpw).astype(jnp.int32)
    idx = (idx + jnp.arange(NW, dtype=jnp.int32)[:, None] * v).reshape(b)
    table = jnp.tile(embedding_table.astype(jnp.float32), (NW, 1))
    out = _build(b, v, d)(idx, table)
    return out.reshape(s, t, d)


# speedup vs baseline: 1.3632x; 1.3632x over previous
import functools

import jax
import jax.numpy as jnp
from jax import lax
from jax.experimental import pallas as pl
from jax.experimental.pallas import tpu as pltpu
from jax.experimental.pallas import tpu_sc as plsc

NC, NS, L = 2, 16, 16
NW = NC * NS
C = 80
G = 48


@functools.lru_cache(maxsize=None)
def _build(B, V, D):
    BPW = B // NW
    NCH = BPW // C
    assert BPW * NW == B and NCH * C == BPW and NCH % 2 == 0
    assert G % L == 0 and (C - G) % L == 0 and D % L == 0
    TPAD = -(-V // 8) * 8

    mesh = plsc.VectorSubcoreMesh(core_axis_name="c", subcore_axis_name="s")

    @functools.partial(
        pl.kernel,
        out_type=jax.ShapeDtypeStruct((B, D), jnp.float32),
        mesh=mesh,
        compiler_params=pltpu.CompilerParams(needs_layout_passes=False),
        scratch_types=[
            pltpu.VMEM((BPW,), jnp.int32),
            pltpu.VMEM((TPAD, D), jnp.float32),
            pltpu.VMEM((C, D), jnp.float32),
            pltpu.VMEM((C, D), jnp.float32),
            pltpu.SemaphoreType.DMA,
            pltpu.SemaphoreType.DMA,
            pltpu.SemaphoreType.DMA,
            pltpu.SemaphoreType.DMA,
        ],
    )
    def emb(idx_hbm, table_hbm, out_hbm, idx_v, table_v, rows0, rows1,
            g0, g1, o0, o1):
        rows = (rows0, rows1)
        gsem = (g0, g1)
        osem = (o0, o1)
        wid = lax.axis_index("s") * NC + lax.axis_index("c")
        base = wid * BPW
        pltpu.sync_copy(idx_hbm.at[pl.ds(base, BPW)], idx_v)
        pltpu.sync_copy(table_hbm.at[pl.ds(0, TPAD)], table_v)

        def wait_g(b):
            pltpu.make_async_copy(table_hbm.at[idx_v.at[pl.ds(0, G)]],
                                  rows[b].at[pl.ds(0, G)], gsem[b]).wait()

        def wait_o(b):
            pltpu.make_async_copy(rows[b], out_hbm.at[pl.ds(0, C)],
                                  osem[b]).wait()

        def step(c, b):
            @pl.when(c >= 2)
            def _():
                wait_o(b)

            pltpu.async_copy(table_hbm.at[idx_v.at[pl.ds(c * C, G)]],
                             rows[b].at[pl.ds(0, G)], gsem[b])

            for g2 in range((C - G) // L):
                idx16 = idx_v[pl.ds(c * C + G + g2 * L, L)]
                rws = [idx16[u] for u in range(L)]

                @plsc.parallel_loop(0, D // L, unroll=D // L)
                def jbody(j, rws=rws, g2=g2, b=b):
                    off = j * L
                    for u in range(L):
                        rows[b][G + g2 * L + u, pl.ds(off, L)] = (
                            table_v[rws[u], pl.ds(off, L)])

            wait_g(b)
            pltpu.async_copy(rows[b], out_hbm.at[pl.ds(base + c * C, C)],
                             osem[b])

        def body(i, carry):
            step(2 * i, 0)
            step(2 * i + 1, 1)
            return carry

        lax.fori_loop(0, NCH // 2, body, 0)
        wait_o(0)
        wait_o(1)

    return emb


def kernel(word_seq, embedding_table):
    s, t = word_seq.shape
    b = s * t
    v, d = embedding_table.shape
    bpw = b // NW
    nch = bpw // C
    idx = word_seq.reshape(NW, nch, C).astype(jnp.int32)
    bias = (jnp.arange(C, dtype=jnp.int32) < G).astype(jnp.int32)[None, None, :]
    bias = bias * (jnp.arange(NW, dtype=jnp.int32) * v)[:, None, None]
    idx = (idx + bias).reshape(b)
    table = jnp.tile(embedding_table.astype(jnp.float32), (NW, 1))
    out = _build(b, v, d)(idx, table)
    return out.reshape(s, t, d)
